# baseline (speedup 1.0000x reference)
import functools

import jax
import jax.numpy as jnp
import numpy as np
from jax import lax
from jax.experimental import pallas as pl
from jax.experimental.pallas import tpu as pltpu
from jax.experimental.pallas import tpu_sc as plsc

_N = 10000
_C = 128
_M = 10000
_K = 32
_H = 8
_CM = 16

_MK = _M * _K
_GC = _C // 2
_RB = 128
_NW = 32
_NBLK = _MK // _RB
_BASE_BLKS = _NBLK // _NW
_EXTRA = _NBLK % _NW

_E = np.kron(np.eye(_H, dtype=np.float32), np.ones((1, _C // _H), np.float32))

_MB = 80
_RWS = _MB * _K


_NB = -(-_NBLK // _NW)


@functools.partial(
    pl.kernel,
    mesh=plsc.VectorSubcoreMesh(core_axis_name="c", subcore_axis_name="s"),
    out_type=jax.ShapeDtypeStruct((_MK, _C), jnp.float32),
    scratch_types=[
        pltpu.VMEM((2, _RB), jnp.int32),
        pltpu.VMEM((2, _RB, _C), jnp.float32),
        pltpu.VMEM_SHARED((_N, _C), jnp.float32),
        pltpu.SemaphoreType.DMA((2,)),
        pltpu.SemaphoreType.DMA((2,)),
        pltpu.SemaphoreType.DMA((2,)),
    ],
)
def _sc_gather(feat_hbm, idx_hbm, out_hbm, idx_v, rows_v, feat_sh,
               isem, gsem, wsem):
    wid = lax.axis_index("s") * 2 + lax.axis_index("c")

    def rbase_of(j):
        g = wid + _NW * j
        return _RB * jnp.where(g < _NBLK, g, wid)

    def icopy(j):
        b = j % 2
        return pltpu.make_async_copy(
            idx_hbm.at[pl.ds(rbase_of(j), _RB)], idx_v.at[b], isem.at[b])

    def gcopy(j):
        b = j % 2
        return pltpu.make_async_copy(
            feat_sh.at[idx_v.at[b]], rows_v.at[b], gsem.at[b])

    def wcopy(j):
        b = j % 2
        return pltpu.make_async_copy(
            rows_v.at[b], out_hbm.at[pl.ds(rbase_of(j), _RB)], wsem.at[b])

    icopy(0).start()
    icopy(1).start()
    @pl.when(lax.axis_index("s") == 0)
    def _():
        pltpu.sync_copy(feat_hbm, feat_sh)
    plsc.subcore_barrier()
    icopy(0).wait()
    gcopy(0).start()

    def body(j, carry):
        gcopy(j).wait()
        wcopy(j).start()

        @pl.when(j >= 1)
        def _():
            wcopy(j - 1).wait()

        @pl.when(j + 1 < _NB)
        def _():
            icopy(j + 1).wait()
            gcopy(j + 1).start()

        @pl.when(j + 2 < _NB)
        def _():
            icopy(j + 2).start()

        return carry

    lax.fori_loop(0, _NB, body, 0)
    wcopy(_NB - 1).wait()


def _tc_body(e_ref, g_ref, guid_ref, w_ref, o_ref):
    guid = guid_ref[...].reshape(_RWS, _H)
    w = w_ref[...].reshape(_RWS, _CM).astype(jnp.bfloat16)
    guid_exp = jnp.dot(guid, e_ref[...],
                       preferred_element_type=jnp.float32
                       ).astype(jnp.bfloat16)
    g = g_ref[...].astype(jnp.bfloat16) * guid_exp
    for p in range(_MB):
        o = lax.dot_general(
            g[p * _K:(p + 1) * _K, :],
            w[p * _K:(p + 1) * _K, :],
            (((0,), (0,)), ((), ())),
            preferred_element_type=jnp.float32)
        for q in range(_CM):
            o_ref[p, q] = o[8 * q:8 * q + 8, :]


_tc_einsum = pl.pallas_call(
    _tc_body,
    grid=(_M // _MB,),
    in_specs=[
        pl.BlockSpec((_H, _C), lambda i: (0, 0)),
        pl.BlockSpec((_RWS, _C), lambda i: (i, 0)),
        pl.BlockSpec((1, _MB, _K, _H), lambda i: (0, i, 0, 0)),
        pl.BlockSpec((1, _MB, _K, _CM), lambda i: (0, i, 0, 0)),
    ],
    out_specs=pl.BlockSpec((_MB, _CM, 8, _CM), lambda i: (i, 0, 0, 0)),
    out_shape=jax.ShapeDtypeStruct((_M, _CM, 8, _CM), jnp.float32),
)


def kernel(input_features, neighbor_inds, guidance, weightnet):
    input_features, neighbor_inds, guidance, weightnet = (
        jax.lax.optimization_barrier(
            (input_features, neighbor_inds, guidance, weightnet)))
    B, N, C = input_features.shape
    _, M, K = neighbor_inds.shape
    feat = input_features.reshape(N, C)
    idx = neighbor_inds.reshape(M * K)
    gathered = _sc_gather(feat, idx)
    out = _tc_einsum(jnp.asarray(_E, jnp.float32), gathered,
                     guidance, weightnet)
    return out.reshape(1, M, C * _CM)


# speedup vs baseline: 1.1810x; 1.1810x over previous
import functools

import jax
import jax.numpy as jnp
import numpy as np
from jax import lax
from jax.experimental import pallas as pl
from jax.experimental.pallas import tpu as pltpu
from jax.experimental.pallas import tpu_sc as plsc

_N = 10000
_C = 128
_M = 10000
_K = 32
_H = 8
_CM = 16

_MK = _M * _K
_RB = 128
_NW = 32
_NBLK = _MK // _RB
_NB = -(-_NBLK // _NW)

_E = np.kron(np.eye(_H, dtype=np.float32), np.ones((1, _C // _H), np.float32))

_MB = 200
_RWS = _MB * _K


@functools.partial(
    pl.kernel,
    mesh=plsc.VectorSubcoreMesh(core_axis_name="c", subcore_axis_name="s"),
    out_type=jax.ShapeDtypeStruct((_MK, _C), jnp.float32),
    scratch_types=[
        pltpu.VMEM((2, _RB), jnp.int32),
        pltpu.VMEM((2, _RB, _C), jnp.float32),
        pltpu.VMEM_SHARED((_N, _C), jnp.float32),
        pltpu.SemaphoreType.DMA((2,)),
        pltpu.SemaphoreType.DMA((2,)),
        pltpu.SemaphoreType.DMA((2,)),
    ],
)
def _sc_gather(feat_hbm, idx_hbm, out_hbm, idx_v, rows_v, feat_sh,
               isem, gsem, wsem):
    wid = lax.axis_index("s") * 2 + lax.axis_index("c")

    def rbase_of(j):
        g = wid + _NW * j
        return _RB * jnp.where(g < _NBLK, g, wid)

    def icopy(j):
        b = j % 2
        return pltpu.make_async_copy(
            idx_hbm.at[pl.ds(rbase_of(j), _RB)], idx_v.at[b], isem.at[b])

    def gcopy(j):
        b = j % 2
        return pltpu.make_async_copy(
            feat_sh.at[idx_v.at[b]], rows_v.at[b], gsem.at[b])

    def wcopy(j):
        b = j % 2
        return pltpu.make_async_copy(
            rows_v.at[b], out_hbm.at[pl.ds(rbase_of(j), _RB)], wsem.at[b])

    icopy(0).start()
    icopy(1).start()
    @pl.when(lax.axis_index("s") == 0)
    def _():
        pltpu.sync_copy(feat_hbm, feat_sh)
    plsc.subcore_barrier()
    icopy(0).wait()
    gcopy(0).start()

    def body(j, carry):
        gcopy(j).wait()
        wcopy(j).start()

        @pl.when(j >= 1)
        def _():
            wcopy(j - 1).wait()

        @pl.when(j + 1 < _NB)
        def _():
            icopy(j + 1).wait()
            gcopy(j + 1).start()

        @pl.when(j + 2 < _NB)
        def _():
            icopy(j + 2).start()

        return carry

    lax.fori_loop(0, _NB, body, 0)
    wcopy(_NB - 1).wait()


def _tc_body(e_ref, g_ref, guid_ref, w_ref, o_ref):
    guid_exp = jnp.dot(guid_ref[...], e_ref[...],
                       preferred_element_type=jnp.float32
                       ).astype(jnp.bfloat16)
    g = g_ref[...].astype(jnp.bfloat16) * guid_exp
    w = w_ref[...]
    for p in range(_MB):
        o_ref[p] = lax.dot_general(
            g[p * _K:(p + 1) * _K, :],
            w[p * _K:(p + 1) * _K, :],
            (((0,), (0,)), ((), ())),
            preferred_element_type=jnp.float32)


_tc_einsum = pl.pallas_call(
    _tc_body,
    grid=(_M // _MB,),
    in_specs=[
        pl.BlockSpec((_H, _C), lambda i: (0, 0)),
        pl.BlockSpec((_RWS, _C), lambda i: (i, 0)),
        pl.BlockSpec((_RWS, _H), lambda i: (i, 0)),
        pl.BlockSpec((_RWS, _CM), lambda i: (i, 0)),
    ],
    out_specs=pl.BlockSpec((_MB, _C, _CM), lambda i: (i, 0, 0)),
    out_shape=jax.ShapeDtypeStruct((_M, _C, _CM), jnp.float32),
)


def kernel(input_features, neighbor_inds, guidance, weightnet):
    B, N, C = input_features.shape
    _, M, K = neighbor_inds.shape
    feat = input_features.reshape(N, C)
    idx = neighbor_inds.reshape(M * K)
    guid = guidance.astype(jnp.bfloat16).reshape(M * K, _H)
    w = weightnet.astype(jnp.bfloat16).reshape(M * K, _CM)
    gathered = _sc_gather(feat, idx)
    out = _tc_einsum(jnp.asarray(_E, jnp.bfloat16), gathered, guid, w)
    return out.reshape(B, M, C * _CM)


# speedup vs baseline: 1.1838x; 1.0024x over previous
import functools

import jax
import jax.numpy as jnp
import numpy as np
from jax import lax
from jax.experimental import pallas as pl
from jax.experimental.pallas import tpu as pltpu
from jax.experimental.pallas import tpu_sc as plsc

_N = 10000
_C = 128
_M = 10000
_K = 32
_H = 8
_CM = 16

_MK = _M * _K
_RB = 128
_NW = 32
_NBLK = _MK // _RB
_NB = -(-_NBLK // _NW)

_E = np.kron(np.eye(_H, dtype=np.float32), np.ones((1, _C // _H), np.float32))

_MB = 250
_RWS = _MB * _K


@functools.partial(
    pl.kernel,
    mesh=plsc.VectorSubcoreMesh(core_axis_name="c", subcore_axis_name="s"),
    out_type=jax.ShapeDtypeStruct((_MK, _C), jnp.float32),
    scratch_types=[
        pltpu.VMEM((2, _RB), jnp.int32),
        pltpu.VMEM((2, _RB, _C), jnp.float32),
        pltpu.VMEM_SHARED((_N, _C), jnp.float32),
        pltpu.SemaphoreType.DMA((2,)),
        pltpu.SemaphoreType.DMA((2,)),
        pltpu.SemaphoreType.DMA((2,)),
    ],
)
def _sc_gather(feat_hbm, idx_hbm, out_hbm, idx_v, rows_v, feat_sh,
               isem, gsem, wsem):
    wid = lax.axis_index("s") * 2 + lax.axis_index("c")

    def rbase_of(j):
        g = wid + _NW * j
        return _RB * jnp.where(g < _NBLK, g, wid)

    def icopy(j):
        b = j % 2
        return pltpu.make_async_copy(
            idx_hbm.at[pl.ds(rbase_of(j), _RB)], idx_v.at[b], isem.at[b])

    def gcopy(j):
        b = j % 2
        return pltpu.make_async_copy(
            feat_sh.at[idx_v.at[b]], rows_v.at[b], gsem.at[b])

    def wcopy(j):
        b = j % 2
        return pltpu.make_async_copy(
            rows_v.at[b], out_hbm.at[pl.ds(rbase_of(j), _RB)], wsem.at[b])

    icopy(0).start()
    icopy(1).start()
    @pl.when(lax.axis_index("s") == 0)
    def _():
        pltpu.sync_copy(feat_hbm, feat_sh)
    plsc.subcore_barrier()
    icopy(0).wait()
    gcopy(0).start()

    def body(j, carry):
        gcopy(j).wait()
        wcopy(j).start()

        @pl.when(j >= 1)
        def _():
            wcopy(j - 1).wait()

        @pl.when(j + 1 < _NB)
        def _():
            icopy(j + 1).wait()
            gcopy(j + 1).start()

        @pl.when(j + 2 < _NB)
        def _():
            icopy(j + 2).start()

        return carry

    lax.fori_loop(0, _NB, body, 0)
    wcopy(_NB - 1).wait()


def _tc_body(e_ref, g_ref, guid_ref, w_ref, o_ref):
    guid_exp = jnp.dot(guid_ref[...], e_ref[...],
                       preferred_element_type=jnp.float32
                       ).astype(jnp.bfloat16)
    g = g_ref[...].astype(jnp.bfloat16) * guid_exp
    w = w_ref[...]
    for p in range(_MB):
        o_ref[p] = lax.dot_general(
            g[p * _K:(p + 1) * _K, :],
            w[p * _K:(p + 1) * _K, :],
            (((0,), (0,)), ((), ())),
            preferred_element_type=jnp.float32)


_tc_einsum = pl.pallas_call(
    _tc_body,
    grid=(_M // _MB,),
    in_specs=[
        pl.BlockSpec((_H, _C), lambda i: (0, 0)),
        pl.BlockSpec((_RWS, _C), lambda i: (i, 0)),
        pl.BlockSpec((_RWS, _H), lambda i: (i, 0)),
        pl.BlockSpec((_RWS, _CM), lambda i: (i, 0)),
    ],
    out_specs=pl.BlockSpec((_MB, _C, _CM), lambda i: (i, 0, 0)),
    out_shape=jax.ShapeDtypeStruct((_M, _C, _CM), jnp.float32),
)


def kernel(input_features, neighbor_inds, guidance, weightnet):
    B, N, C = input_features.shape
    _, M, K = neighbor_inds.shape
    feat = input_features.reshape(N, C)
    idx = neighbor_inds.reshape(M * K)
    guid = guidance.astype(jnp.bfloat16).reshape(M * K, _H)
    w = weightnet.astype(jnp.bfloat16).reshape(M * K, _CM)
    gathered = _sc_gather(feat, idx)
    out = _tc_einsum(jnp.asarray(_E, jnp.bfloat16), gathered, guid, w)
    return out.reshape(B, M, C * _CM)
